# baseline (speedup 1.0000x reference)
import functools

import jax
import jax.numpy as jnp
from jax import lax
from jax.experimental import pallas as pl
from jax.experimental.pallas import tpu as pltpu
from jax.experimental.pallas import tpu_sc as plsc

N = 50000
E = 3200000
H = 8
B = 16
NC = 2
NS = 16
NW = NC * NS
CHUNK = 128
SUP = 16
EPS = SUP * CHUNK
EROWS = E // CHUNK
RN = N // 16
TSLICE = 3128
LAST_START = (NS - 1) * TSLICE
LAST_SIZE = N - LAST_START

_MESH = plsc.VectorSubcoreMesh(core_axis_name="c", subcore_axis_name="s")
_SC_PARAMS = pltpu.CompilerParams(use_tc_tiling_on_sc=False)

_SUPS_TOTAL = EROWS // SUP
_TAIL_ROWS = EROWS % SUP
assert _TAIL_ROWS % 8 == 0
_SQ, _SR = divmod(_SUPS_TOTAL, NW)
_TAIL_START = _SUPS_TOTAL * SUP


def _edge_supers(wid):
    sup0 = wid * _SQ + jnp.minimum(wid, _SR)
    nsup = _SQ + (wid < _SR).astype(jnp.int32)
    return sup0, nsup


def _tile_slices(s, body):
    @pl.when(s < NS - 1)
    def _():
        body(s * TSLICE, TSLICE)

    @pl.when(s == NS - 1)
    def _():
        body(LAST_START, LAST_SIZE)


def _stage_in(hbm, hbm_base, sp, stage, s):
    def body(start, size):
        pltpu.sync_copy(hbm.at[pl.ds(hbm_base + start, size)],
                        stage.at[pl.ds(0, size)])
        pltpu.sync_copy(stage.at[pl.ds(0, size)],
                        sp.at[pl.ds(start, size)])
    _tile_slices(s, body)


def _stage_out(sp, hbm, hbm_base, stage, s):
    def body(start, size):
        pltpu.sync_copy(sp.at[pl.ds(start, size)],
                        stage.at[pl.ds(0, size)])
        pltpu.sync_copy(stage.at[pl.ds(0, size)],
                        hbm.at[pl.ds(hbm_base + start, size)])
    _tile_slices(s, body)


def _make_deg_kernel():
    @functools.partial(
        pl.kernel,
        out_type=jax.ShapeDtypeStruct((NC * N,), jnp.float32),
        mesh=_MESH,
        compiler_params=_SC_PARAMS,
        scratch_types=[
            pltpu.VMEM_SHARED((N,), jnp.float32),
            pltpu.VMEM((TSLICE,), jnp.float32),
            pltpu.VMEM((2, EPS), jnp.int32),
            pltpu.VMEM((EPS,), jnp.float32),
            pltpu.SemaphoreType.DMA,
            pltpu.SemaphoreType.DMA,
        ],
    )
    def deg_kernel(dst_hbm, zeros_hbm, deg_hbm, acc, stage, dbuf, ones,
                   isem, ssem):
        c = lax.axis_index("c")
        s = lax.axis_index("s")
        wid = c * NS + s

        for i in range(EPS // 16):
            ones[pl.ds(i * 16, 16)] = jnp.ones((16,), jnp.float32)

        _stage_in(zeros_hbm, 0, acc, stage, s)
        plsc.subcore_barrier()

        sup0, nsup = _edge_supers(wid)

        def fire_idx(g, p):
            base = (sup0 + g) * EPS
            pltpu.async_copy(dst_hbm.at[pl.ds(base, EPS)], dbuf.at[p], isem)

        def wait_idx(p):
            pltpu.make_async_copy(dst_hbm.at[pl.ds(0, EPS)], dbuf.at[p],
                                  isem).wait()

        def drain_scatters():
            pltpu.make_async_copy(zeros_hbm.at[pl.ds(0, EPS)], ones,
                                  ssem).wait()

        fire_idx(0, 0)

        def sup_body(g, carry):
            p = lax.rem(g, 2)
            q = 1 - p
            wait_idx(p)

            @pl.when(g > 0)
            def _():
                drain_scatters()

            @pl.when(g + 1 < nsup)
            def _():
                fire_idx(g + 1, q)

            pltpu.async_copy(ones, acc.at[dbuf.at[p]], ssem, add=True)
            return carry

        lax.fori_loop(0, nsup, sup_body, 0)
        drain_scatters()

        if _TAIL_ROWS:
            @pl.when(wid == NW - 1)
            def _():
                tn = _TAIL_ROWS * CHUNK
                pltpu.sync_copy(dst_hbm.at[pl.ds(_TAIL_START * CHUNK, tn)],
                                dbuf.at[0].at[pl.ds(0, tn)])
                pltpu.async_copy(
                    ones.at[pl.ds(0, tn)],
                    acc.at[dbuf.at[0].at[pl.ds(0, tn)]], ssem,
                    add=True).wait()

        plsc.subcore_barrier()
        _stage_out(acc, deg_hbm, c * N, stage, s)

    return deg_kernel


def _make_agg_kernel(width):
    tshape = (N,) if width == 1 else (N, width)
    sshape = (TSLICE,) if width == 1 else (TSLICE, width)
    rshape = (2, EPS) if width == 1 else (2, EPS, width)
    zshape = (NC * N,) if width == 1 else (NC * N, width)

    @functools.partial(
        pl.kernel,
        out_type=jax.ShapeDtypeStruct(zshape, jnp.float32),
        mesh=_MESH,
        compiler_params=_SC_PARAMS,
        scratch_types=[
            pltpu.VMEM_SHARED(tshape, jnp.float32),
            pltpu.VMEM(sshape, jnp.float32),
            pltpu.VMEM((2, EPS), jnp.int32),
            pltpu.VMEM((2, EPS), jnp.int32),
            pltpu.VMEM(rshape, jnp.float32),
            pltpu.SemaphoreType.DMA,
            pltpu.SemaphoreType.DMA,
            pltpu.SemaphoreType.DMA,
        ],
    )
    def agg_kernel(u_hbm, src_hbm, dst_hbm, zeros_hbm, z_hbm,
                   acc, stage, sbuf, dbuf, rows, isem, gsem, ssem):
        c = lax.axis_index("c")
        s = lax.axis_index("s")
        wid = c * NS + s

        _stage_in(zeros_hbm, 0, acc, stage, s)
        plsc.subcore_barrier()

        sup0, nsup = _edge_supers(wid)
        dummy = u_hbm.at[pl.ds(0, EPS)]

        def fire_idx(g, p):
            base = (sup0 + g) * EPS
            pltpu.async_copy(src_hbm.at[pl.ds(base, EPS)], sbuf.at[p], isem)
            pltpu.async_copy(dst_hbm.at[pl.ds(base, EPS)], dbuf.at[p], isem)

        def wait_idx(p):
            pltpu.make_async_copy(src_hbm.at[pl.ds(0, EPS)], sbuf.at[p],
                                  isem).wait()
            pltpu.make_async_copy(dst_hbm.at[pl.ds(0, EPS)], dbuf.at[p],
                                  isem).wait()

        def drain_scatters(p):
            pltpu.make_async_copy(dummy, rows.at[p], ssem).wait()

        fire_idx(0, 0)

        def sup_body(g, carry):
            p = lax.rem(g, 2)
            q = 1 - p
            wait_idx(p)
            gd = pltpu.async_copy(u_hbm.at[sbuf.at[p]], rows.at[p], gsem)

            @pl.when(g > 0)
            def _():
                drain_scatters(q)

            @pl.when(g + 1 < nsup)
            def _():
                fire_idx(g + 1, q)

            gd.wait()
            pltpu.async_copy(rows.at[p], acc.at[dbuf.at[p]], ssem, add=True)
            return carry

        lax.fori_loop(0, nsup, sup_body, 0)
        drain_scatters(lax.rem(nsup - 1, 2))

        if _TAIL_ROWS:
            @pl.when(wid == NW - 1)
            def _():
                pltpu.sync_copy(src_hbm.at[pl.ds(_TAIL_START, _TAIL_ROWS)],
                                sbuf.at[0].at[pl.ds(0, _TAIL_ROWS)])
                pltpu.sync_copy(dst_hbm.at[pl.ds(_TAIL_START, _TAIL_ROWS)],
                                dbuf.at[0].at[pl.ds(0, _TAIL_ROWS)])
                tsl = pl.ds(0, _TAIL_ROWS)
                pltpu.async_copy(u_hbm.at[sbuf.at[0].at[tsl]],
                                 rows.at[0].at[tsl], gsem).wait()
                pltpu.async_copy(rows.at[0].at[tsl],
                                 acc.at[dbuf.at[0].at[tsl]], ssem,
                                 add=True).wait()

        plsc.subcore_barrier()
        _stage_out(acc, z_hbm, c * N, stage, s)

    return agg_kernel


_deg_kernel = _make_deg_kernel()
_agg1_kernel = _make_agg_kernel(1)
_agg8_kernel = _make_agg_kernel(H)


def _t0_body(deg_ref, x_ref, e16_ref, u1_ref, dinv16_ref, dinv8_ref):
    degv = deg_ref[0] + deg_ref[1] + 1.0
    dinv = lax.rsqrt(degv)
    dinv16_ref[...] = dinv
    u1_ref[...] = dinv * x_ref[...]
    dinv8_ref[...] = jnp.dot(dinv, e16_ref[...],
                             preferred_element_type=jnp.float32,
                 precision=lax.Precision.HIGHEST)


def _t1_body(z_ref, u1_ref, dinv16_ref, dinv8_ref, w_ref, b_ref, e16_ref,
             u2_ref):
    y = dinv16_ref[...] * (z_ref[0] + z_ref[1] + u1_ref[...])
    yb = jnp.dot(y, e16_ref[...], preferred_element_type=jnp.float32,
                 precision=lax.Precision.HIGHEST)
    u2_ref[...] = dinv8_ref[...] * jnp.maximum(
        yb * w_ref[...] + b_ref[...], 0.0)


def _tmid_body(z_ref, u_ref, dinv8_ref, wd_ref, b_ref, unext_ref):
    y = dinv8_ref[...] * (z_ref[0] + z_ref[1] + u_ref[...])
    h = jnp.dot(y, wd_ref[...], preferred_element_type=jnp.float32,
                 precision=lax.Precision.HIGHEST) \
        + b_ref[...]
    unext_ref[...] = dinv8_ref[...] * jnp.maximum(h, 0.0)


def _t4_body(z_ref, u_ref, dinv8_ref, wd_ref, b_ref, idx_ref, e16_ref,
             g8_ref, out_ref):
    y = dinv8_ref[...] * (z_ref[0] + z_ref[1] + u_ref[...])
    h = jnp.dot(y, wd_ref[...], preferred_element_type=jnp.float32,
                 precision=lax.Precision.HIGHEST) \
        + b_ref[...]
    idx = idx_ref[...]
    r = idx // 16
    col = idx % 16
    rowsel = (lax.broadcasted_iota(jnp.int32, (B, RN), 1)
              == r).astype(jnp.float32)
    rows = jnp.dot(rowsel, h, preferred_element_type=jnp.float32,
                 precision=lax.Precision.HIGHEST)
    csel = (lax.broadcasted_iota(jnp.int32, (B, 16), 1)
            == col).astype(jnp.float32)
    cexp = jnp.dot(csel, e16_ref[...],
                   preferred_element_type=jnp.float32,
                 precision=lax.Precision.HIGHEST)
    out_ref[...] = jnp.dot(rows * cexp, g8_ref[...],
                           preferred_element_type=jnp.float32,
                 precision=lax.Precision.HIGHEST)


_f32 = jnp.float32


def kernel(x, edge_index, indices, W1, b1, W2, b2, W3, b3, W4, b4):
    src = edge_index[0]
    dst = edge_index[1]
    zeros1 = jnp.zeros((N,), _f32)
    zeros8 = jnp.zeros((N, H), _f32)

    e16 = (jnp.arange(128)[None, :] // H
           == jnp.arange(16)[:, None]).astype(_f32)
    g8 = jnp.tile(jnp.eye(H, dtype=_f32), (16, 1))
    w128_1 = jnp.tile(W1[0], 16).reshape(1, 128)
    b128 = [jnp.tile(b, 16).reshape(1, 128) for b in (b1, b2, b3, b4)]
    eye16 = jnp.eye(16, dtype=_f32)
    wd2 = jnp.kron(eye16, W2)
    wd3 = jnp.kron(eye16, W3)
    wd4 = jnp.kron(eye16, W4)

    deg = _deg_kernel(dst, zeros1)

    u1, dinv16, dinv8 = pl.pallas_call(
        _t0_body,
        out_shape=[
            jax.ShapeDtypeStruct((RN, 16), _f32),
            jax.ShapeDtypeStruct((RN, 16), _f32),
            jax.ShapeDtypeStruct((RN, 128), _f32),
        ],
    )(deg.reshape(NC, RN, 16), x.reshape(RN, 16), e16)

    z1 = _agg1_kernel(u1.reshape(N), src, dst, zeros1)

    u2 = pl.pallas_call(
        _t1_body,
        out_shape=jax.ShapeDtypeStruct((RN, 128), _f32),
    )(z1.reshape(NC, RN, 16), u1, dinv16, dinv8, w128_1, b128[0], e16)

    z2 = _agg8_kernel(u2.reshape(N, H), src, dst, zeros8)

    u3 = pl.pallas_call(
        _tmid_body,
        out_shape=jax.ShapeDtypeStruct((RN, 128), _f32),
    )(z2.reshape(NC, RN, 128), u2, dinv8, wd2, b128[1])

    z3 = _agg8_kernel(u3.reshape(N, H), src, dst, zeros8)

    u4 = pl.pallas_call(
        _tmid_body,
        out_shape=jax.ShapeDtypeStruct((RN, 128), _f32),
    )(z3.reshape(NC, RN, 128), u3, dinv8, wd3, b128[2])

    z4 = _agg8_kernel(u4.reshape(N, H), src, dst, zeros8)

    return pl.pallas_call(
        _t4_body,
        out_shape=jax.ShapeDtypeStruct((B, H), _f32),
    )(z4.reshape(NC, RN, 128), u4, dinv8, wd4, b128[3],
      indices.reshape(B, 1), e16, g8)


# speedup vs baseline: 1.5860x; 1.5860x over previous
import functools

import jax
import jax.numpy as jnp
from jax import lax
from jax.experimental import pallas as pl
from jax.experimental.pallas import tpu as pltpu
from jax.experimental.pallas import tpu_sc as plsc

N = 50000
E = 3200000
H = 8
B = 16
NC = 2
NS = 16
NW = NC * NS
CHUNK = 128
SUP = 16
EPS = SUP * CHUNK
EROWS = E // CHUNK
RN = N // 16
TSLICE = 3128
LAST_START = (NS - 1) * TSLICE
LAST_SIZE = N - LAST_START

_MESH = plsc.VectorSubcoreMesh(core_axis_name="c", subcore_axis_name="s")
_SC_PARAMS = pltpu.CompilerParams(use_tc_tiling_on_sc=False)

_SUPS_TOTAL = EROWS // SUP
_TAIL_ROWS = EROWS % SUP
assert _TAIL_ROWS % 8 == 0
_SQ, _SR = divmod(_SUPS_TOTAL, NW)
_TAIL_START = _SUPS_TOTAL * SUP


def _edge_supers(wid):
    sup0 = wid * _SQ + jnp.minimum(wid, _SR)
    nsup = _SQ + (wid < _SR).astype(jnp.int32)
    return sup0, nsup


def _tile_slices(s, body):
    @pl.when(s < NS - 1)
    def _():
        body(s * TSLICE, TSLICE)

    @pl.when(s == NS - 1)
    def _():
        body(LAST_START, LAST_SIZE)


def _stage_in(hbm, hbm_base, sp, stage, s):
    def body(start, size):
        pltpu.sync_copy(hbm.at[pl.ds(hbm_base + start, size)],
                        stage.at[pl.ds(0, size)])
        pltpu.sync_copy(stage.at[pl.ds(0, size)],
                        sp.at[pl.ds(start, size)])
    _tile_slices(s, body)


def _stage_out(sp, hbm, hbm_base, stage, s):
    def body(start, size):
        pltpu.sync_copy(sp.at[pl.ds(start, size)],
                        stage.at[pl.ds(0, size)])
        pltpu.sync_copy(stage.at[pl.ds(0, size)],
                        hbm.at[pl.ds(hbm_base + start, size)])
    _tile_slices(s, body)


def _make_deg_kernel():
    @functools.partial(
        pl.kernel,
        out_type=jax.ShapeDtypeStruct((NC * N,), jnp.float32),
        mesh=_MESH,
        compiler_params=_SC_PARAMS,
        scratch_types=[
            pltpu.VMEM_SHARED((N,), jnp.float32),
            pltpu.VMEM((TSLICE,), jnp.float32),
            pltpu.VMEM((2, EPS), jnp.int32),
            pltpu.VMEM((EPS,), jnp.float32),
            pltpu.SemaphoreType.DMA,
            pltpu.SemaphoreType.DMA,
        ],
    )
    def deg_kernel(dst_hbm, zeros_hbm, deg_hbm, acc, stage, dbuf, ones,
                   isem, ssem):
        c = lax.axis_index("c")
        s = lax.axis_index("s")
        wid = c * NS + s

        for i in range(EPS // 16):
            ones[pl.ds(i * 16, 16)] = jnp.ones((16,), jnp.float32)

        _stage_in(zeros_hbm, 0, acc, stage, s)
        plsc.subcore_barrier()

        sup0, nsup = _edge_supers(wid)

        def fire_idx(g, p):
            base = (sup0 + g) * EPS
            pltpu.async_copy(dst_hbm.at[pl.ds(base, EPS)], dbuf.at[p], isem)

        def wait_idx(p):
            pltpu.make_async_copy(dst_hbm.at[pl.ds(0, EPS)], dbuf.at[p],
                                  isem).wait()

        def drain_scatters():
            pltpu.make_async_copy(zeros_hbm.at[pl.ds(0, EPS)], ones,
                                  ssem).wait()

        fire_idx(0, 0)

        def sup_body(g, carry):
            p = lax.rem(g, 2)
            q = 1 - p
            wait_idx(p)

            @pl.when(g > 0)
            def _():
                drain_scatters()

            @pl.when(g + 1 < nsup)
            def _():
                fire_idx(g + 1, q)

            pltpu.async_copy(ones, acc.at[dbuf.at[p]], ssem, add=True)
            return carry

        lax.fori_loop(0, nsup, sup_body, 0)
        drain_scatters()

        if _TAIL_ROWS:
            @pl.when(wid == NW - 1)
            def _():
                tn = _TAIL_ROWS * CHUNK
                pltpu.sync_copy(dst_hbm.at[pl.ds(_TAIL_START * CHUNK, tn)],
                                dbuf.at[0].at[pl.ds(0, tn)])
                pltpu.async_copy(
                    ones.at[pl.ds(0, tn)],
                    acc.at[dbuf.at[0].at[pl.ds(0, tn)]], ssem,
                    add=True).wait()

        plsc.subcore_barrier()
        _stage_out(acc, deg_hbm, c * N, stage, s)

    return deg_kernel


def _make_agg_kernel(width):
    tshape = (N,) if width == 1 else (N, width)
    sshape = (TSLICE,) if width == 1 else (TSLICE, width)
    rshape = (2, EPS) if width == 1 else (2, EPS, width)
    zshape = (NC * N,) if width == 1 else (NC * N, width)

    @functools.partial(
        pl.kernel,
        out_type=jax.ShapeDtypeStruct(zshape, jnp.float32),
        mesh=_MESH,
        compiler_params=_SC_PARAMS,
        scratch_types=[
            pltpu.VMEM_SHARED(tshape, jnp.float32),
            pltpu.VMEM_SHARED(tshape, jnp.float32),
            pltpu.VMEM(sshape, jnp.float32),
            pltpu.VMEM((2, EPS), jnp.int32),
            pltpu.VMEM((2, EPS), jnp.int32),
            pltpu.VMEM(rshape, jnp.float32),
            pltpu.SemaphoreType.DMA,
            pltpu.SemaphoreType.DMA,
            pltpu.SemaphoreType.DMA,
        ],
    )
    def agg_kernel(u_hbm, src_hbm, dst_hbm, zeros_hbm, z_hbm,
                   tab, acc, stage, sbuf, dbuf, rows, isem, gsem, ssem):
        c = lax.axis_index("c")
        s = lax.axis_index("s")
        wid = c * NS + s

        _stage_in(u_hbm, 0, tab, stage, s)
        _stage_in(zeros_hbm, 0, acc, stage, s)
        plsc.subcore_barrier()

        sup0, nsup = _edge_supers(wid)
        dummy = u_hbm.at[pl.ds(0, EPS)]

        def fire_idx(g, p):
            base = (sup0 + g) * EPS
            pltpu.async_copy(src_hbm.at[pl.ds(base, EPS)], sbuf.at[p], isem)
            pltpu.async_copy(dst_hbm.at[pl.ds(base, EPS)], dbuf.at[p], isem)

        def wait_idx(p):
            pltpu.make_async_copy(src_hbm.at[pl.ds(0, EPS)], sbuf.at[p],
                                  isem).wait()
            pltpu.make_async_copy(dst_hbm.at[pl.ds(0, EPS)], dbuf.at[p],
                                  isem).wait()

        def drain_scatters(p):
            pltpu.make_async_copy(dummy, rows.at[p], ssem).wait()

        fire_idx(0, 0)

        def sup_body(g, carry):
            p = lax.rem(g, 2)
            q = 1 - p
            wait_idx(p)
            gd = pltpu.async_copy(tab.at[sbuf.at[p]], rows.at[p], gsem)

            @pl.when(g > 0)
            def _():
                drain_scatters(q)

            @pl.when(g + 1 < nsup)
            def _():
                fire_idx(g + 1, q)

            gd.wait()
            pltpu.async_copy(rows.at[p], acc.at[dbuf.at[p]], ssem, add=True)
            return carry

        lax.fori_loop(0, nsup, sup_body, 0)
        drain_scatters(lax.rem(nsup - 1, 2))

        if _TAIL_ROWS:
            @pl.when(wid == NW - 1)
            def _():
                pltpu.sync_copy(src_hbm.at[pl.ds(_TAIL_START, _TAIL_ROWS)],
                                sbuf.at[0].at[pl.ds(0, _TAIL_ROWS)])
                pltpu.sync_copy(dst_hbm.at[pl.ds(_TAIL_START, _TAIL_ROWS)],
                                dbuf.at[0].at[pl.ds(0, _TAIL_ROWS)])
                tsl = pl.ds(0, _TAIL_ROWS)
                pltpu.async_copy(tab.at[sbuf.at[0].at[tsl]],
                                 rows.at[0].at[tsl], gsem).wait()
                pltpu.async_copy(rows.at[0].at[tsl],
                                 acc.at[dbuf.at[0].at[tsl]], ssem,
                                 add=True).wait()

        plsc.subcore_barrier()
        _stage_out(acc, z_hbm, c * N, stage, s)

    return agg_kernel


_deg_kernel = _make_deg_kernel()
_agg1_kernel = _make_agg_kernel(1)
_agg8_kernel = _make_agg_kernel(H)


def _t0_body(deg_ref, x_ref, e16_ref, u1_ref, dinv16_ref, dinv8_ref):
    degv = deg_ref[0] + deg_ref[1] + 1.0
    dinv = lax.rsqrt(degv)
    dinv16_ref[...] = dinv
    u1_ref[...] = dinv * x_ref[...]
    dinv8_ref[...] = jnp.dot(dinv, e16_ref[...],
                             preferred_element_type=jnp.float32,
                 precision=lax.Precision.HIGHEST)


def _t1_body(z_ref, u1_ref, dinv16_ref, dinv8_ref, w_ref, b_ref, e16_ref,
             u2_ref):
    y = dinv16_ref[...] * (z_ref[0] + z_ref[1] + u1_ref[...])
    yb = jnp.dot(y, e16_ref[...], preferred_element_type=jnp.float32,
                 precision=lax.Precision.HIGHEST)
    u2_ref[...] = dinv8_ref[...] * jnp.maximum(
        yb * w_ref[...] + b_ref[...], 0.0)


def _tmid_body(z_ref, u_ref, dinv8_ref, wd_ref, b_ref, unext_ref):
    y = dinv8_ref[...] * (z_ref[0] + z_ref[1] + u_ref[...])
    h = jnp.dot(y, wd_ref[...], preferred_element_type=jnp.float32,
                 precision=lax.Precision.HIGHEST) \
        + b_ref[...]
    unext_ref[...] = dinv8_ref[...] * jnp.maximum(h, 0.0)


def _t4_body(z_ref, u_ref, dinv8_ref, wd_ref, b_ref, idx_ref, e16_ref,
             g8_ref, out_ref):
    y = dinv8_ref[...] * (z_ref[0] + z_ref[1] + u_ref[...])
    h = jnp.dot(y, wd_ref[...], preferred_element_type=jnp.float32,
                 precision=lax.Precision.HIGHEST) \
        + b_ref[...]
    idx = idx_ref[...]
    r = idx // 16
    col = idx % 16
    rowsel = (lax.broadcasted_iota(jnp.int32, (B, RN), 1)
              == r).astype(jnp.float32)
    rows = jnp.dot(rowsel, h, preferred_element_type=jnp.float32,
                 precision=lax.Precision.HIGHEST)
    csel = (lax.broadcasted_iota(jnp.int32, (B, 16), 1)
            == col).astype(jnp.float32)
    cexp = jnp.dot(csel, e16_ref[...],
                   preferred_element_type=jnp.float32,
                 precision=lax.Precision.HIGHEST)
    out_ref[...] = jnp.dot(rows * cexp, g8_ref[...],
                           preferred_element_type=jnp.float32,
                 precision=lax.Precision.HIGHEST)


_f32 = jnp.float32


def kernel(x, edge_index, indices, W1, b1, W2, b2, W3, b3, W4, b4):
    src = edge_index[0]
    dst = edge_index[1]
    zeros1 = jnp.zeros((N,), _f32)
    zeros8 = jnp.zeros((N, H), _f32)

    e16 = (jnp.arange(128)[None, :] // H
           == jnp.arange(16)[:, None]).astype(_f32)
    g8 = jnp.tile(jnp.eye(H, dtype=_f32), (16, 1))
    w128_1 = jnp.tile(W1[0], 16).reshape(1, 128)
    b128 = [jnp.tile(b, 16).reshape(1, 128) for b in (b1, b2, b3, b4)]
    eye16 = jnp.eye(16, dtype=_f32)
    wd2 = jnp.kron(eye16, W2)
    wd3 = jnp.kron(eye16, W3)
    wd4 = jnp.kron(eye16, W4)

    deg = _deg_kernel(dst, zeros1)

    u1, dinv16, dinv8 = pl.pallas_call(
        _t0_body,
        out_shape=[
            jax.ShapeDtypeStruct((RN, 16), _f32),
            jax.ShapeDtypeStruct((RN, 16), _f32),
            jax.ShapeDtypeStruct((RN, 128), _f32),
        ],
    )(deg.reshape(NC, RN, 16), x.reshape(RN, 16), e16)

    z1 = _agg1_kernel(u1.reshape(N), src, dst, zeros1)

    u2 = pl.pallas_call(
        _t1_body,
        out_shape=jax.ShapeDtypeStruct((RN, 128), _f32),
    )(z1.reshape(NC, RN, 16), u1, dinv16, dinv8, w128_1, b128[0], e16)

    z2 = _agg8_kernel(u2.reshape(N, H), src, dst, zeros8)

    u3 = pl.pallas_call(
        _tmid_body,
        out_shape=jax.ShapeDtypeStruct((RN, 128), _f32),
    )(z2.reshape(NC, RN, 128), u2, dinv8, wd2, b128[1])

    z3 = _agg8_kernel(u3.reshape(N, H), src, dst, zeros8)

    u4 = pl.pallas_call(
        _tmid_body,
        out_shape=jax.ShapeDtypeStruct((RN, 128), _f32),
    )(z3.reshape(NC, RN, 128), u3, dinv8, wd3, b128[2])

    z4 = _agg8_kernel(u4.reshape(N, H), src, dst, zeros8)

    return pl.pallas_call(
        _t4_body,
        out_shape=jax.ShapeDtypeStruct((B, H), _f32),
    )(z4.reshape(NC, RN, 128), u4, dinv8, wd4, b128[3],
      indices.reshape(B, 1), e16, g8)


# speedup vs baseline: 1.6132x; 1.0172x over previous
import functools

import jax
import jax.numpy as jnp
from jax import lax
from jax.experimental import pallas as pl
from jax.experimental.pallas import tpu as pltpu
from jax.experimental.pallas import tpu_sc as plsc

N = 50000
E = 3200000
H = 8
B = 16
NC = 2
NS = 16
NW = NC * NS
CHUNK = 128
EROWS = E // CHUNK
RN = N // 16
TSLICE = 3128
LAST_START = (NS - 1) * TSLICE
LAST_SIZE = N - LAST_START

_MESH = plsc.VectorSubcoreMesh(core_axis_name="c", subcore_axis_name="s")
_SC_PARAMS = pltpu.CompilerParams(use_tc_tiling_on_sc=False)

class _EdgePlan:
    def __init__(self, sup):
        self.sup = sup
        self.eps = sup * CHUNK
        self.sups_total = EROWS // sup
        self.tail_rows = EROWS % sup
        assert self.tail_rows % 8 == 0
        self.sq, self.sr = divmod(self.sups_total, NW)
        self.tail_start = self.sups_total * sup

    def edge_supers(self, wid):
        sup0 = wid * self.sq + jnp.minimum(wid, self.sr)
        nsup = self.sq + (wid < self.sr).astype(jnp.int32)
        return sup0, nsup


def _tile_slices(s, body):
    @pl.when(s < NS - 1)
    def _():
        body(s * TSLICE, TSLICE)

    @pl.when(s == NS - 1)
    def _():
        body(LAST_START, LAST_SIZE)


def _stage_in(hbm, hbm_base, sp, stage, s):
    def body(start, size):
        pltpu.sync_copy(hbm.at[pl.ds(hbm_base + start, size)],
                        stage.at[pl.ds(0, size)])
        pltpu.sync_copy(stage.at[pl.ds(0, size)],
                        sp.at[pl.ds(start, size)])
    _tile_slices(s, body)


def _stage_out(sp, hbm, hbm_base, stage, s):
    def body(start, size):
        pltpu.sync_copy(sp.at[pl.ds(start, size)],
                        stage.at[pl.ds(0, size)])
        pltpu.sync_copy(stage.at[pl.ds(0, size)],
                        hbm.at[pl.ds(hbm_base + start, size)])
    _tile_slices(s, body)


def _make_deg_kernel():
    plan = _EdgePlan(32)
    EPS = plan.eps
    @functools.partial(
        pl.kernel,
        out_type=jax.ShapeDtypeStruct((NC * N,), jnp.float32),
        mesh=_MESH,
        compiler_params=_SC_PARAMS,
        scratch_types=[
            pltpu.VMEM_SHARED((N,), jnp.float32),
            pltpu.VMEM((TSLICE,), jnp.float32),
            pltpu.VMEM((2, EPS), jnp.int32),
            pltpu.VMEM((EPS,), jnp.float32),
            pltpu.SemaphoreType.DMA,
            pltpu.SemaphoreType.DMA,
        ],
    )
    def deg_kernel(dst_hbm, zeros_hbm, deg_hbm, acc, stage, dbuf, ones,
                   isem, ssem):
        c = lax.axis_index("c")
        s = lax.axis_index("s")
        wid = c * NS + s

        for i in range(EPS // 16):
            ones[pl.ds(i * 16, 16)] = jnp.ones((16,), jnp.float32)

        _stage_in(zeros_hbm, 0, acc, stage, s)
        plsc.subcore_barrier()

        sup0, nsup = plan.edge_supers(wid)

        def fire_idx(g, p):
            base = (sup0 + g) * EPS
            pltpu.async_copy(dst_hbm.at[pl.ds(base, EPS)], dbuf.at[p], isem)

        def wait_idx(p):
            pltpu.make_async_copy(dst_hbm.at[pl.ds(0, EPS)], dbuf.at[p],
                                  isem).wait()

        def drain_scatters():
            pltpu.make_async_copy(zeros_hbm.at[pl.ds(0, EPS)], ones,
                                  ssem).wait()

        fire_idx(0, 0)

        def sup_body(g, carry):
            p = lax.rem(g, 2)
            q = 1 - p
            wait_idx(p)

            @pl.when(g > 0)
            def _():
                drain_scatters()

            @pl.when(g + 1 < nsup)
            def _():
                fire_idx(g + 1, q)

            pltpu.async_copy(ones, acc.at[dbuf.at[p]], ssem, add=True)
            return carry

        lax.fori_loop(0, nsup, sup_body, 0)
        drain_scatters()

        if plan.tail_rows:
            @pl.when(wid == NW - 1)
            def _():
                tn = plan.tail_rows * CHUNK
                pltpu.sync_copy(dst_hbm.at[pl.ds(plan.tail_start * CHUNK, tn)],
                                dbuf.at[0].at[pl.ds(0, tn)])
                pltpu.async_copy(
                    ones.at[pl.ds(0, tn)],
                    acc.at[dbuf.at[0].at[pl.ds(0, tn)]], ssem,
                    add=True).wait()

        plsc.subcore_barrier()
        _stage_out(acc, deg_hbm, c * N, stage, s)

    return deg_kernel


def _make_agg_kernel(width):
    plan = _EdgePlan(32 if width == 1 else 16)
    EPS = plan.eps
    tshape = (N,) if width == 1 else (N, width)
    sshape = (TSLICE,) if width == 1 else (TSLICE, width)
    rshape = (2, EPS) if width == 1 else (2, EPS, width)
    zshape = (NC * N,) if width == 1 else (NC * N, width)

    @functools.partial(
        pl.kernel,
        out_type=jax.ShapeDtypeStruct(zshape, jnp.float32),
        mesh=_MESH,
        compiler_params=_SC_PARAMS,
        scratch_types=[
            pltpu.VMEM_SHARED(tshape, jnp.float32),
            pltpu.VMEM_SHARED(tshape, jnp.float32),
            pltpu.VMEM(sshape, jnp.float32),
            pltpu.VMEM((2, EPS), jnp.int32),
            pltpu.VMEM((2, EPS), jnp.int32),
            pltpu.VMEM(rshape, jnp.float32),
            pltpu.SemaphoreType.DMA,
            pltpu.SemaphoreType.DMA,
            pltpu.SemaphoreType.DMA,
        ],
    )
    def agg_kernel(u_hbm, src_hbm, dst_hbm, zeros_hbm, z_hbm,
                   tab, acc, stage, sbuf, dbuf, rows, isem, gsem, ssem):
        c = lax.axis_index("c")
        s = lax.axis_index("s")
        wid = c * NS + s

        _stage_in(u_hbm, 0, tab, stage, s)
        _stage_in(zeros_hbm, 0, acc, stage, s)
        plsc.subcore_barrier()

        sup0, nsup = plan.edge_supers(wid)
        dummy = u_hbm.at[pl.ds(0, EPS)]

        def fire_idx(g, p):
            base = (sup0 + g) * EPS
            pltpu.async_copy(src_hbm.at[pl.ds(base, EPS)], sbuf.at[p], isem)
            pltpu.async_copy(dst_hbm.at[pl.ds(base, EPS)], dbuf.at[p], isem)

        def wait_idx(p):
            pltpu.make_async_copy(src_hbm.at[pl.ds(0, EPS)], sbuf.at[p],
                                  isem).wait()
            pltpu.make_async_copy(dst_hbm.at[pl.ds(0, EPS)], dbuf.at[p],
                                  isem).wait()

        def drain_scatters(p):
            pltpu.make_async_copy(dummy, rows.at[p], ssem).wait()

        fire_idx(0, 0)

        def sup_body(g, carry):
            p = lax.rem(g, 2)
            q = 1 - p
            wait_idx(p)
            gd = pltpu.async_copy(tab.at[sbuf.at[p]], rows.at[p], gsem)

            @pl.when(g > 0)
            def _():
                drain_scatters(q)

            @pl.when(g + 1 < nsup)
            def _():
                fire_idx(g + 1, q)

            gd.wait()
            pltpu.async_copy(rows.at[p], acc.at[dbuf.at[p]], ssem, add=True)
            return carry

        lax.fori_loop(0, nsup, sup_body, 0)
        drain_scatters(lax.rem(nsup - 1, 2))

        if plan.tail_rows:
            @pl.when(wid == NW - 1)
            def _():
                tn = plan.tail_rows * CHUNK
                tsl = pl.ds(0, tn)
                pltpu.sync_copy(
                    src_hbm.at[pl.ds(plan.tail_start * CHUNK, tn)],
                    sbuf.at[0].at[tsl])
                pltpu.sync_copy(
                    dst_hbm.at[pl.ds(plan.tail_start * CHUNK, tn)],
                    dbuf.at[0].at[tsl])
                pltpu.async_copy(tab.at[sbuf.at[0].at[tsl]],
                                 rows.at[0].at[tsl], gsem).wait()
                pltpu.async_copy(rows.at[0].at[tsl],
                                 acc.at[dbuf.at[0].at[tsl]], ssem,
                                 add=True).wait()

        plsc.subcore_barrier()
        _stage_out(acc, z_hbm, c * N, stage, s)

    return agg_kernel


_deg_kernel = _make_deg_kernel()
_agg1_kernel = _make_agg_kernel(1)
_agg8_kernel = _make_agg_kernel(H)


def _t0_body(deg_ref, x_ref, e16_ref, u1_ref, dinv16_ref, dinv8_ref):
    degv = deg_ref[0] + deg_ref[1] + 1.0
    dinv = lax.rsqrt(degv)
    dinv16_ref[...] = dinv
    u1_ref[...] = dinv * x_ref[...]
    dinv8_ref[...] = jnp.dot(dinv, e16_ref[...],
                             preferred_element_type=jnp.float32,
                 precision=lax.Precision.HIGHEST)


def _t1_body(z_ref, u1_ref, dinv16_ref, dinv8_ref, w_ref, b_ref, e16_ref,
             u2_ref):
    y = dinv16_ref[...] * (z_ref[0] + z_ref[1] + u1_ref[...])
    yb = jnp.dot(y, e16_ref[...], preferred_element_type=jnp.float32,
                 precision=lax.Precision.HIGHEST)
    u2_ref[...] = dinv8_ref[...] * jnp.maximum(
        yb * w_ref[...] + b_ref[...], 0.0)


def _tmid_body(z_ref, u_ref, dinv8_ref, wd_ref, b_ref, unext_ref):
    y = dinv8_ref[...] * (z_ref[0] + z_ref[1] + u_ref[...])
    h = jnp.dot(y, wd_ref[...], preferred_element_type=jnp.float32,
                 precision=lax.Precision.HIGHEST) \
        + b_ref[...]
    unext_ref[...] = dinv8_ref[...] * jnp.maximum(h, 0.0)


def _t4_body(z_ref, u_ref, dinv8_ref, wd_ref, b_ref, idx_ref, e16_ref,
             g8_ref, out_ref):
    y = dinv8_ref[...] * (z_ref[0] + z_ref[1] + u_ref[...])
    h = jnp.dot(y, wd_ref[...], preferred_element_type=jnp.float32,
                 precision=lax.Precision.HIGHEST) \
        + b_ref[...]
    idx = idx_ref[...]
    r = idx // 16
    col = idx % 16
    rowsel = (lax.broadcasted_iota(jnp.int32, (B, RN), 1)
              == r).astype(jnp.float32)
    rows = jnp.dot(rowsel, h, preferred_element_type=jnp.float32,
                 precision=lax.Precision.HIGHEST)
    csel = (lax.broadcasted_iota(jnp.int32, (B, 16), 1)
            == col).astype(jnp.float32)
    cexp = jnp.dot(csel, e16_ref[...],
                   preferred_element_type=jnp.float32,
                 precision=lax.Precision.HIGHEST)
    out_ref[...] = jnp.dot(rows * cexp, g8_ref[...],
                           preferred_element_type=jnp.float32,
                 precision=lax.Precision.HIGHEST)


_f32 = jnp.float32


def kernel(x, edge_index, indices, W1, b1, W2, b2, W3, b3, W4, b4):
    src = edge_index[0]
    dst = edge_index[1]
    zeros1 = jnp.zeros((N,), _f32)
    zeros8 = jnp.zeros((N, H), _f32)

    e16 = (jnp.arange(128)[None, :] // H
           == jnp.arange(16)[:, None]).astype(_f32)
    g8 = jnp.tile(jnp.eye(H, dtype=_f32), (16, 1))
    w128_1 = jnp.tile(W1[0], 16).reshape(1, 128)
    b128 = [jnp.tile(b, 16).reshape(1, 128) for b in (b1, b2, b3, b4)]
    eye16 = jnp.eye(16, dtype=_f32)
    wd2 = jnp.kron(eye16, W2)
    wd3 = jnp.kron(eye16, W3)
    wd4 = jnp.kron(eye16, W4)

    deg = _deg_kernel(dst, zeros1)

    u1, dinv16, dinv8 = pl.pallas_call(
        _t0_body,
        out_shape=[
            jax.ShapeDtypeStruct((RN, 16), _f32),
            jax.ShapeDtypeStruct((RN, 16), _f32),
            jax.ShapeDtypeStruct((RN, 128), _f32),
        ],
    )(deg.reshape(NC, RN, 16), x.reshape(RN, 16), e16)

    z1 = _agg1_kernel(u1.reshape(N), src, dst, zeros1)

    u2 = pl.pallas_call(
        _t1_body,
        out_shape=jax.ShapeDtypeStruct((RN, 128), _f32),
    )(z1.reshape(NC, RN, 16), u1, dinv16, dinv8, w128_1, b128[0], e16)

    z2 = _agg8_kernel(u2.reshape(N, H), src, dst, zeros8)

    u3 = pl.pallas_call(
        _tmid_body,
        out_shape=jax.ShapeDtypeStruct((RN, 128), _f32),
    )(z2.reshape(NC, RN, 128), u2, dinv8, wd2, b128[1])

    z3 = _agg8_kernel(u3.reshape(N, H), src, dst, zeros8)

    u4 = pl.pallas_call(
        _tmid_body,
        out_shape=jax.ShapeDtypeStruct((RN, 128), _f32),
    )(z3.reshape(NC, RN, 128), u3, dinv8, wd3, b128[2])

    z4 = _agg8_kernel(u4.reshape(N, H), src, dst, zeros8)

    return pl.pallas_call(
        _t4_body,
        out_shape=jax.ShapeDtypeStruct((B, H), _f32),
    )(z4.reshape(NC, RN, 128), u4, dinv8, wd4, b128[3],
      indices.reshape(B, 1), e16, g8)


# speedup vs baseline: 1.6879x; 1.0463x over previous
import functools

import jax
import jax.numpy as jnp
from jax import lax
from jax.experimental import pallas as pl
from jax.experimental.pallas import tpu as pltpu
from jax.experimental.pallas import tpu_sc as plsc

N = 50000
E = 3200000
H = 8
B = 16
NC = 2
NS = 16
NW = NC * NS
CHUNK = 128
EROWS = E // CHUNK
RN = N // 16
TSLICE = 3128
LAST_START = (NS - 1) * TSLICE
LAST_SIZE = N - LAST_START
SHALF = 1568

_MESH = plsc.VectorSubcoreMesh(core_axis_name="c", subcore_axis_name="s")
_SC_PARAMS = pltpu.CompilerParams(use_tc_tiling_on_sc=False)

class _EdgePlan:
    def __init__(self, sup):
        self.sup = sup
        self.eps = sup * CHUNK
        self.sups_total = EROWS // sup
        self.tail_rows = EROWS % sup
        assert self.tail_rows % 8 == 0
        self.sq, self.sr = divmod(self.sups_total, NW)
        self.tail_start = self.sups_total * sup

    def edge_supers(self, wid):
        sup0 = wid * self.sq + jnp.minimum(wid, self.sr)
        nsup = self.sq + (wid < self.sr).astype(jnp.int32)
        return sup0, nsup


def _tile_slices(s, body):
    @pl.when(s < NS - 1)
    def _():
        body(s * TSLICE, TSLICE)

    @pl.when(s == NS - 1)
    def _():
        body(LAST_START, LAST_SIZE)


def _halves(start, size):
    return ((start, SHALF), (start + SHALF, size - SHALF))


def _stage_in(hbm, hbm_base, sp, stage, s):
    def body(start, size):
        for st, sz in _halves(start, size):
            pltpu.sync_copy(hbm.at[pl.ds(hbm_base + st, sz)],
                            stage.at[pl.ds(0, sz)])
            pltpu.sync_copy(stage.at[pl.ds(0, sz)],
                            sp.at[pl.ds(st, sz)])
    _tile_slices(s, body)


def _stage_out(sp, hbm, hbm_base, stage, s):
    def body(start, size):
        for st, sz in _halves(start, size):
            pltpu.sync_copy(sp.at[pl.ds(st, sz)],
                            stage.at[pl.ds(0, sz)])
            pltpu.sync_copy(stage.at[pl.ds(0, sz)],
                            hbm.at[pl.ds(hbm_base + st, sz)])
    _tile_slices(s, body)


def _make_deg_kernel():
    plan = _EdgePlan(32)
    EPS = plan.eps
    @functools.partial(
        pl.kernel,
        out_type=jax.ShapeDtypeStruct((NC * N,), jnp.float32),
        mesh=_MESH,
        compiler_params=_SC_PARAMS,
        scratch_types=[
            pltpu.VMEM_SHARED((N,), jnp.float32),
            pltpu.VMEM((SHALF,), jnp.float32),
            pltpu.VMEM((3, EPS), jnp.int32),
            pltpu.VMEM((EPS,), jnp.float32),
            pltpu.SemaphoreType.DMA,
            pltpu.SemaphoreType.DMA,
        ],
    )
    def deg_kernel(dst_hbm, zeros_hbm, deg_hbm, acc, stage, dbuf, ones,
                   isem, ssem):
        c = lax.axis_index("c")
        s = lax.axis_index("s")
        wid = c * NS + s

        for i in range(EPS // 16):
            ones[pl.ds(i * 16, 16)] = jnp.ones((16,), jnp.float32)

        _stage_in(zeros_hbm, 0, acc, stage, s)
        plsc.subcore_barrier()

        sup0, nsup = plan.edge_supers(wid)

        def fire_idx(g, p):
            base = (sup0 + g) * EPS
            pltpu.async_copy(dst_hbm.at[pl.ds(base, EPS)], dbuf.at[p], isem)

        def wait_idx(p):
            pltpu.make_async_copy(dst_hbm.at[pl.ds(0, EPS)], dbuf.at[p],
                                  isem).wait()

        def drain_scatters():
            pltpu.make_async_copy(zeros_hbm.at[pl.ds(0, EPS)], ones,
                                  ssem).wait()

        fire_idx(0, 0)

        def sup_body(g, carry):
            a = lax.rem(g, 3)
            ap1 = lax.rem(g + 1, 3)

            @pl.when(g > 1)
            def _():
                drain_scatters()

            wait_idx(a)

            @pl.when(g + 1 < nsup)
            def _():
                fire_idx(g + 1, ap1)

            pltpu.async_copy(ones, acc.at[dbuf.at[a]], ssem, add=True)
            return carry

        lax.fori_loop(0, nsup, sup_body, 0)
        drain_scatters()
        drain_scatters()

        if plan.tail_rows:
            @pl.when(wid == NW - 1)
            def _():
                tn = plan.tail_rows * CHUNK
                pltpu.sync_copy(dst_hbm.at[pl.ds(plan.tail_start * CHUNK, tn)],
                                dbuf.at[0].at[pl.ds(0, tn)])
                pltpu.async_copy(
                    ones.at[pl.ds(0, tn)],
                    acc.at[dbuf.at[0].at[pl.ds(0, tn)]], ssem,
                    add=True).wait()

        plsc.subcore_barrier()
        _stage_out(acc, deg_hbm, c * N, stage, s)

    return deg_kernel


def _make_agg_kernel(width):
    plan = _EdgePlan(32 if width == 1 else 16)
    EPS = plan.eps
    tshape = (N,) if width == 1 else (N, width)
    sshape = (SHALF,) if width == 1 else (SHALF, width)
    rshape = (3, EPS) if width == 1 else (3, EPS, width)
    zshape = (NC * N,) if width == 1 else (NC * N, width)

    @functools.partial(
        pl.kernel,
        out_type=jax.ShapeDtypeStruct(zshape, jnp.float32),
        mesh=_MESH,
        compiler_params=_SC_PARAMS,
        scratch_types=[
            pltpu.VMEM_SHARED(tshape, jnp.float32),
            pltpu.VMEM_SHARED(tshape, jnp.float32),
            pltpu.VMEM(sshape, jnp.float32),
            pltpu.VMEM((3, EPS), jnp.int32),
            pltpu.VMEM((3, EPS), jnp.int32),
            pltpu.VMEM(rshape, jnp.float32),
            pltpu.SemaphoreType.DMA,
            pltpu.SemaphoreType.DMA,
            pltpu.SemaphoreType.DMA,
        ],
    )
    def agg_kernel(u_hbm, src_hbm, dst_hbm, zeros_hbm, z_hbm,
                   tab, acc, stage, sbuf, dbuf, rows, isem, gsem, ssem):
        c = lax.axis_index("c")
        s = lax.axis_index("s")
        wid = c * NS + s

        _stage_in(u_hbm, 0, tab, stage, s)
        _stage_in(zeros_hbm, 0, acc, stage, s)
        plsc.subcore_barrier()

        sup0, nsup = plan.edge_supers(wid)
        dummy = u_hbm.at[pl.ds(0, EPS)]

        def fire_idx(g, p):
            base = (sup0 + g) * EPS
            pltpu.async_copy(src_hbm.at[pl.ds(base, EPS)], sbuf.at[p], isem)
            pltpu.async_copy(dst_hbm.at[pl.ds(base, EPS)], dbuf.at[p], isem)

        def wait_idx(p):
            pltpu.make_async_copy(src_hbm.at[pl.ds(0, EPS)], sbuf.at[p],
                                  isem).wait()
            pltpu.make_async_copy(dst_hbm.at[pl.ds(0, EPS)], dbuf.at[p],
                                  isem).wait()

        def drain_gather(p):
            pltpu.make_async_copy(dummy, rows.at[p], gsem).wait()

        def drain_scatter(p):
            pltpu.make_async_copy(dummy, rows.at[p], ssem).wait()

        def fire_scatter(p):
            pltpu.async_copy(rows.at[p], acc.at[dbuf.at[p]], ssem, add=True)

        fire_idx(0, 0)

        def sup_body(g, carry):
            a = lax.rem(g, 3)
            am1 = lax.rem(g + 2, 3)
            am2 = lax.rem(g + 1, 3)

            @pl.when(g > 0)
            def _():
                drain_gather(am1)
                fire_scatter(am1)

            @pl.when(g > 1)
            def _():
                drain_scatter(am2)

            wait_idx(a)

            @pl.when(g + 1 < nsup)
            def _():
                fire_idx(g + 1, am2)

            pltpu.async_copy(tab.at[sbuf.at[a]], rows.at[a], gsem)
            return carry

        lax.fori_loop(0, nsup, sup_body, 0)
        last = lax.rem(nsup + 2, 3)
        last2 = lax.rem(nsup + 1, 3)
        drain_gather(last)
        fire_scatter(last)
        drain_scatter(last2)
        drain_scatter(last)

        if plan.tail_rows:
            @pl.when(wid == NW - 1)
            def _():
                tn = plan.tail_rows * CHUNK
                tsl = pl.ds(0, tn)
                pltpu.sync_copy(
                    src_hbm.at[pl.ds(plan.tail_start * CHUNK, tn)],
                    sbuf.at[0].at[tsl])
                pltpu.sync_copy(
                    dst_hbm.at[pl.ds(plan.tail_start * CHUNK, tn)],
                    dbuf.at[0].at[tsl])
                pltpu.async_copy(tab.at[sbuf.at[0].at[tsl]],
                                 rows.at[0].at[tsl], gsem).wait()
                pltpu.async_copy(rows.at[0].at[tsl],
                                 acc.at[dbuf.at[0].at[tsl]], ssem,
                                 add=True).wait()

        plsc.subcore_barrier()
        _stage_out(acc, z_hbm, c * N, stage, s)

    return agg_kernel


_deg_kernel = _make_deg_kernel()
_agg1_kernel = _make_agg_kernel(1)
_agg8_kernel = _make_agg_kernel(H)


def _t0_body(deg_ref, x_ref, e16_ref, u1_ref, dinv16_ref, dinv8_ref):
    degv = deg_ref[0] + deg_ref[1] + 1.0
    dinv = lax.rsqrt(degv)
    dinv16_ref[...] = dinv
    u1_ref[...] = dinv * x_ref[...]
    dinv8_ref[...] = jnp.dot(dinv, e16_ref[...],
                             preferred_element_type=jnp.float32,
                 precision=lax.Precision.HIGHEST)


def _t1_body(z_ref, u1_ref, dinv16_ref, dinv8_ref, w_ref, b_ref, e16_ref,
             u2_ref):
    y = dinv16_ref[...] * (z_ref[0] + z_ref[1] + u1_ref[...])
    yb = jnp.dot(y, e16_ref[...], preferred_element_type=jnp.float32,
                 precision=lax.Precision.HIGHEST)
    u2_ref[...] = dinv8_ref[...] * jnp.maximum(
        yb * w_ref[...] + b_ref[...], 0.0)


def _tmid_body(z_ref, u_ref, dinv8_ref, wd_ref, b_ref, unext_ref):
    y = dinv8_ref[...] * (z_ref[0] + z_ref[1] + u_ref[...])
    h = jnp.dot(y, wd_ref[...], preferred_element_type=jnp.float32,
                 precision=lax.Precision.HIGHEST) \
        + b_ref[...]
    unext_ref[...] = dinv8_ref[...] * jnp.maximum(h, 0.0)


def _t4_body(z_ref, u_ref, dinv8_ref, wd_ref, b_ref, idx_ref, e16_ref,
             g8_ref, out_ref):
    y = dinv8_ref[...] * (z_ref[0] + z_ref[1] + u_ref[...])
    h = jnp.dot(y, wd_ref[...], preferred_element_type=jnp.float32,
                 precision=lax.Precision.HIGHEST) \
        + b_ref[...]
    idx = idx_ref[...]
    r = idx // 16
    col = idx % 16
    rowsel = (lax.broadcasted_iota(jnp.int32, (B, RN), 1)
              == r).astype(jnp.float32)
    rows = jnp.dot(rowsel, h, preferred_element_type=jnp.float32,
                 precision=lax.Precision.HIGHEST)
    csel = (lax.broadcasted_iota(jnp.int32, (B, 16), 1)
            == col).astype(jnp.float32)
    cexp = jnp.dot(csel, e16_ref[...],
                   preferred_element_type=jnp.float32,
                 precision=lax.Precision.HIGHEST)
    out_ref[...] = jnp.dot(rows * cexp, g8_ref[...],
                           preferred_element_type=jnp.float32,
                 precision=lax.Precision.HIGHEST)


_f32 = jnp.float32


def kernel(x, edge_index, indices, W1, b1, W2, b2, W3, b3, W4, b4):
    src = edge_index[0]
    dst = edge_index[1]
    zeros1 = jnp.zeros((N,), _f32)
    zeros8 = jnp.zeros((N, H), _f32)

    e16 = (jnp.arange(128)[None, :] // H
           == jnp.arange(16)[:, None]).astype(_f32)
    g8 = jnp.tile(jnp.eye(H, dtype=_f32), (16, 1))
    w128_1 = jnp.tile(W1[0], 16).reshape(1, 128)
    b128 = [jnp.tile(b, 16).reshape(1, 128) for b in (b1, b2, b3, b4)]
    eye16 = jnp.eye(16, dtype=_f32)
    wd2 = jnp.kron(eye16, W2)
    wd3 = jnp.kron(eye16, W3)
    wd4 = jnp.kron(eye16, W4)

    deg = _deg_kernel(dst, zeros1)

    u1, dinv16, dinv8 = pl.pallas_call(
        _t0_body,
        out_shape=[
            jax.ShapeDtypeStruct((RN, 16), _f32),
            jax.ShapeDtypeStruct((RN, 16), _f32),
            jax.ShapeDtypeStruct((RN, 128), _f32),
        ],
    )(deg.reshape(NC, RN, 16), x.reshape(RN, 16), e16)

    z1 = _agg1_kernel(u1.reshape(N), src, dst, zeros1)

    u2 = pl.pallas_call(
        _t1_body,
        out_shape=jax.ShapeDtypeStruct((RN, 128), _f32),
    )(z1.reshape(NC, RN, 16), u1, dinv16, dinv8, w128_1, b128[0], e16)

    z2 = _agg8_kernel(u2.reshape(N, H), src, dst, zeros8)

    u3 = pl.pallas_call(
        _tmid_body,
        out_shape=jax.ShapeDtypeStruct((RN, 128), _f32),
    )(z2.reshape(NC, RN, 128), u2, dinv8, wd2, b128[1])

    z3 = _agg8_kernel(u3.reshape(N, H), src, dst, zeros8)

    u4 = pl.pallas_call(
        _tmid_body,
        out_shape=jax.ShapeDtypeStruct((RN, 128), _f32),
    )(z3.reshape(NC, RN, 128), u3, dinv8, wd3, b128[2])

    z4 = _agg8_kernel(u4.reshape(N, H), src, dst, zeros8)

    return pl.pallas_call(
        _t4_body,
        out_shape=jax.ShapeDtypeStruct((B, H), _f32),
    )(z4.reshape(NC, RN, 128), u4, dinv8, wd4, b128[3],
      indices.reshape(B, 1), e16, g8)


# speedup vs baseline: 1.6881x; 1.0001x over previous
import functools

import jax
import jax.numpy as jnp
from jax import lax
from jax.experimental import pallas as pl
from jax.experimental.pallas import tpu as pltpu
from jax.experimental.pallas import tpu_sc as plsc

N = 50000
E = 3200000
H = 8
B = 16
NC = 2
NS = 16
NW = NC * NS
CHUNK = 128
EROWS = E // CHUNK
RN = N // 16
RNP = 3128
NP = RNP * 16
TSLICE = 3128
LAST_START = (NS - 1) * TSLICE
LAST_SIZE = N - LAST_START
SHALF = 1568

_MESH = plsc.VectorSubcoreMesh(core_axis_name="c", subcore_axis_name="s")
_SC_PARAMS = pltpu.CompilerParams(use_tc_tiling_on_sc=False)

class _EdgePlan:
    def __init__(self, sup):
        self.sup = sup
        self.eps = sup * CHUNK
        self.sups_total = EROWS // sup
        self.tail_rows = EROWS % sup
        assert self.tail_rows % 8 == 0
        self.sq, self.sr = divmod(self.sups_total, NW)
        self.tail_start = self.sups_total * sup

    def edge_supers(self, wid):
        sup0 = wid * self.sq + jnp.minimum(wid, self.sr)
        nsup = self.sq + (wid < self.sr).astype(jnp.int32)
        return sup0, nsup


def _tile_slices(s, body):
    @pl.when(s < NS - 1)
    def _():
        body(s * TSLICE, TSLICE)

    @pl.when(s == NS - 1)
    def _():
        body(LAST_START, LAST_SIZE)


def _halves(start, size):
    return ((start, SHALF), (start + SHALF, size - SHALF))


def _stage_in(hbm, hbm_base, sp, stage, s):
    def body(start, size):
        for st, sz in _halves(start, size):
            pltpu.sync_copy(hbm.at[pl.ds(hbm_base + st, sz)],
                            stage.at[pl.ds(0, sz)])
            pltpu.sync_copy(stage.at[pl.ds(0, sz)],
                            sp.at[pl.ds(st, sz)])
    _tile_slices(s, body)


def _stage_out(sp, hbm, hbm_base, stage, s):
    def body(start, size):
        for st, sz in _halves(start, size):
            pltpu.sync_copy(sp.at[pl.ds(st, sz)],
                            stage.at[pl.ds(0, sz)])
            pltpu.sync_copy(stage.at[pl.ds(0, sz)],
                            hbm.at[pl.ds(hbm_base + st, sz)])
    _tile_slices(s, body)


def _make_deg_kernel():
    plan = _EdgePlan(32)
    EPS = plan.eps
    @functools.partial(
        pl.kernel,
        out_type=jax.ShapeDtypeStruct((NC * N,), jnp.float32),
        mesh=_MESH,
        compiler_params=_SC_PARAMS,
        scratch_types=[
            pltpu.VMEM_SHARED((N,), jnp.float32),
            pltpu.VMEM((SHALF,), jnp.float32),
            pltpu.VMEM((3, EPS), jnp.int32),
            pltpu.VMEM((EPS,), jnp.float32),
            pltpu.SemaphoreType.DMA,
            pltpu.SemaphoreType.DMA,
        ],
    )
    def deg_kernel(dst_hbm, zeros_hbm, deg_hbm, acc, stage, dbuf, ones,
                   isem, ssem):
        c = lax.axis_index("c")
        s = lax.axis_index("s")
        wid = c * NS + s

        for i in range(EPS // 16):
            ones[pl.ds(i * 16, 16)] = jnp.ones((16,), jnp.float32)

        _stage_in(zeros_hbm, 0, acc, stage, s)
        plsc.subcore_barrier()

        sup0, nsup = plan.edge_supers(wid)

        def fire_idx(g, p):
            base = (sup0 + g) * EPS
            pltpu.async_copy(dst_hbm.at[pl.ds(base, EPS)], dbuf.at[p], isem)

        def wait_idx(p):
            pltpu.make_async_copy(dst_hbm.at[pl.ds(0, EPS)], dbuf.at[p],
                                  isem).wait()

        def drain_scatters():
            pltpu.make_async_copy(zeros_hbm.at[pl.ds(0, EPS)], ones,
                                  ssem).wait()

        fire_idx(0, 0)

        def sup_body(g, carry):
            a = lax.rem(g, 3)
            ap1 = lax.rem(g + 1, 3)

            @pl.when(g > 1)
            def _():
                drain_scatters()

            wait_idx(a)

            @pl.when(g + 1 < nsup)
            def _():
                fire_idx(g + 1, ap1)

            pltpu.async_copy(ones, acc.at[dbuf.at[a]], ssem, add=True)
            return carry

        lax.fori_loop(0, nsup, sup_body, 0)
        drain_scatters()
        drain_scatters()

        if plan.tail_rows:
            @pl.when(wid == NW - 1)
            def _():
                tn = plan.tail_rows * CHUNK
                pltpu.sync_copy(dst_hbm.at[pl.ds(plan.tail_start * CHUNK, tn)],
                                dbuf.at[0].at[pl.ds(0, tn)])
                pltpu.async_copy(
                    ones.at[pl.ds(0, tn)],
                    acc.at[dbuf.at[0].at[pl.ds(0, tn)]], ssem,
                    add=True).wait()

        plsc.subcore_barrier()
        _stage_out(acc, deg_hbm, c * N, stage, s)

    return deg_kernel


def _make_agg_kernel(width):
    plan = _EdgePlan(32 if width == 1 else 16)
    EPS = plan.eps
    tshape = (N,) if width == 1 else (N, width)
    sshape = (SHALF,) if width == 1 else (SHALF, width)
    rshape = (3, EPS) if width == 1 else (3, EPS, width)
    zshape = (NC * N,) if width == 1 else (NC * N, width)

    @functools.partial(
        pl.kernel,
        out_type=jax.ShapeDtypeStruct(zshape, jnp.float32),
        mesh=_MESH,
        compiler_params=_SC_PARAMS,
        scratch_types=[
            pltpu.VMEM_SHARED(tshape, jnp.float32),
            pltpu.VMEM_SHARED(tshape, jnp.float32),
            pltpu.VMEM(sshape, jnp.float32),
            pltpu.VMEM((3, EPS), jnp.int32),
            pltpu.VMEM((3, EPS), jnp.int32),
            pltpu.VMEM(rshape, jnp.float32),
            pltpu.SemaphoreType.DMA,
            pltpu.SemaphoreType.DMA,
            pltpu.SemaphoreType.DMA,
        ],
    )
    def agg_kernel(u_hbm, src_hbm, dst_hbm, zeros_hbm, z_hbm,
                   tab, acc, stage, sbuf, dbuf, rows, isem, gsem, ssem):
        c = lax.axis_index("c")
        s = lax.axis_index("s")
        wid = c * NS + s

        _stage_in(u_hbm, 0, tab, stage, s)
        _stage_in(zeros_hbm, 0, acc, stage, s)
        plsc.subcore_barrier()

        sup0, nsup = plan.edge_supers(wid)
        dummy = u_hbm.at[pl.ds(0, EPS)]

        def fire_idx(g, p):
            base = (sup0 + g) * EPS
            pltpu.async_copy(src_hbm.at[pl.ds(base, EPS)], sbuf.at[p], isem)
            pltpu.async_copy(dst_hbm.at[pl.ds(base, EPS)], dbuf.at[p], isem)

        def wait_idx(p):
            pltpu.make_async_copy(src_hbm.at[pl.ds(0, EPS)], sbuf.at[p],
                                  isem).wait()
            pltpu.make_async_copy(dst_hbm.at[pl.ds(0, EPS)], dbuf.at[p],
                                  isem).wait()

        def drain_gather(p):
            pltpu.make_async_copy(dummy, rows.at[p], gsem).wait()

        def drain_scatter(p):
            pltpu.make_async_copy(dummy, rows.at[p], ssem).wait()

        def fire_scatter(p):
            pltpu.async_copy(rows.at[p], acc.at[dbuf.at[p]], ssem, add=True)

        fire_idx(0, 0)

        def sup_body(g, carry):
            a = lax.rem(g, 3)
            am1 = lax.rem(g + 2, 3)
            am2 = lax.rem(g + 1, 3)

            @pl.when(g > 0)
            def _():
                drain_gather(am1)
                fire_scatter(am1)

            @pl.when(g > 1)
            def _():
                drain_scatter(am2)

            wait_idx(a)

            @pl.when(g + 1 < nsup)
            def _():
                fire_idx(g + 1, am2)

            pltpu.async_copy(tab.at[sbuf.at[a]], rows.at[a], gsem)
            return carry

        lax.fori_loop(0, nsup, sup_body, 0)
        last = lax.rem(nsup + 2, 3)
        last2 = lax.rem(nsup + 1, 3)
        drain_gather(last)
        fire_scatter(last)
        drain_scatter(last2)
        drain_scatter(last)

        if plan.tail_rows:
            @pl.when(wid == NW - 1)
            def _():
                tn = plan.tail_rows * CHUNK
                tsl = pl.ds(0, tn)
                pltpu.sync_copy(
                    src_hbm.at[pl.ds(plan.tail_start * CHUNK, tn)],
                    sbuf.at[0].at[tsl])
                pltpu.sync_copy(
                    dst_hbm.at[pl.ds(plan.tail_start * CHUNK, tn)],
                    dbuf.at[0].at[tsl])
                pltpu.async_copy(tab.at[sbuf.at[0].at[tsl]],
                                 rows.at[0].at[tsl], gsem).wait()
                pltpu.async_copy(rows.at[0].at[tsl],
                                 acc.at[dbuf.at[0].at[tsl]], ssem,
                                 add=True).wait()

        plsc.subcore_barrier()
        _stage_out(acc, z_hbm, c * N, stage, s)

    return agg_kernel


_deg_kernel = _make_deg_kernel()
_agg1_kernel = _make_agg_kernel(1)
_agg8_kernel = _make_agg_kernel(H)


def _t0_body(deg_ref, x_ref, e16_ref, u1_ref, dinv16_ref, dinv8_ref):
    degv = deg_ref[0] + deg_ref[1] + 1.0
    dinv = lax.rsqrt(degv)
    dinv16_ref[...] = dinv
    u1_ref[...] = dinv * x_ref[...]
    dinv8_ref[...] = jnp.dot(dinv, e16_ref[...],
                             preferred_element_type=jnp.float32,
                 precision=lax.Precision.HIGHEST)


def _t1_body(z_ref, u1_ref, dinv16_ref, dinv8_ref, w_ref, b_ref, e16_ref,
             u2_ref):
    y = dinv16_ref[...] * (z_ref[0] + z_ref[1] + u1_ref[...])
    yb = jnp.dot(y, e16_ref[...], preferred_element_type=jnp.float32,
                 precision=lax.Precision.HIGHEST)
    u2_ref[0:RN, :] = dinv8_ref[...] * jnp.maximum(
        yb * w_ref[...] + b_ref[...], 0.0)


def _tmid_body(z_ref, u_ref, dinv8_ref, wd_ref, b_ref, unext_ref):
    y = dinv8_ref[...] * (z_ref[0] + z_ref[1] + u_ref[0:RN, :])
    h = jnp.dot(y, wd_ref[...], preferred_element_type=jnp.float32,
                 precision=lax.Precision.HIGHEST) \
        + b_ref[...]
    unext_ref[0:RN, :] = dinv8_ref[...] * jnp.maximum(h, 0.0)


def _t4_body(z_ref, u_ref, dinv8_ref, wd_ref, b_ref, idx_ref, e16_ref,
             g8_ref, out_ref):
    y = dinv8_ref[...] * (z_ref[0] + z_ref[1] + u_ref[0:RN, :])
    h = jnp.dot(y, wd_ref[...], preferred_element_type=jnp.float32,
                 precision=lax.Precision.HIGHEST) \
        + b_ref[...]
    idx = idx_ref[...]
    r = idx // 16
    col = idx % 16
    rowsel = (lax.broadcasted_iota(jnp.int32, (B, RN), 1)
              == r).astype(jnp.float32)
    rows = jnp.dot(rowsel, h, preferred_element_type=jnp.float32,
                 precision=lax.Precision.HIGHEST)
    csel = (lax.broadcasted_iota(jnp.int32, (B, 16), 1)
            == col).astype(jnp.float32)
    cexp = jnp.dot(csel, e16_ref[...],
                   preferred_element_type=jnp.float32,
                 precision=lax.Precision.HIGHEST)
    out_ref[...] = jnp.dot(rows * cexp, g8_ref[...],
                           preferred_element_type=jnp.float32,
                 precision=lax.Precision.HIGHEST)


_f32 = jnp.float32


def kernel(x, edge_index, indices, W1, b1, W2, b2, W3, b3, W4, b4):
    src = edge_index[0]
    dst = edge_index[1]
    zeros1 = jnp.zeros((N,), _f32)
    zeros8 = jnp.zeros((N, H), _f32)

    e16 = (jnp.arange(128)[None, :] // H
           == jnp.arange(16)[:, None]).astype(_f32)
    g8 = jnp.tile(jnp.eye(H, dtype=_f32), (16, 1))
    w128_1 = jnp.tile(W1[0], 16).reshape(1, 128)
    b128 = [jnp.tile(b, 16).reshape(1, 128) for b in (b1, b2, b3, b4)]
    eye16 = jnp.eye(16, dtype=_f32)
    wd2 = jnp.kron(eye16, W2)
    wd3 = jnp.kron(eye16, W3)
    wd4 = jnp.kron(eye16, W4)

    deg = _deg_kernel(dst, zeros1)

    u1, dinv16, dinv8 = pl.pallas_call(
        _t0_body,
        out_shape=[
            jax.ShapeDtypeStruct((RN, 16), _f32),
            jax.ShapeDtypeStruct((RN, 16), _f32),
            jax.ShapeDtypeStruct((RN, 128), _f32),
        ],
    )(deg.reshape(NC, RN, 16), x.reshape(RN, 16), e16)

    z1 = _agg1_kernel(u1.reshape(N), src, dst, zeros1)

    u2 = pl.pallas_call(
        _t1_body,
        out_shape=jax.ShapeDtypeStruct((RNP, 128), _f32),
    )(z1.reshape(NC, RN, 16), u1, dinv16, dinv8, w128_1, b128[0], e16)

    z2 = _agg8_kernel(u2.reshape(NP, H), src, dst, zeros8)

    u3 = pl.pallas_call(
        _tmid_body,
        out_shape=jax.ShapeDtypeStruct((RNP, 128), _f32),
    )(z2.reshape(NC, RN, 128), u2, dinv8, wd2, b128[1])

    z3 = _agg8_kernel(u3.reshape(NP, H), src, dst, zeros8)

    u4 = pl.pallas_call(
        _tmid_body,
        out_shape=jax.ShapeDtypeStruct((RNP, 128), _f32),
    )(z3.reshape(NC, RN, 128), u3, dinv8, wd3, b128[2])

    z4 = _agg8_kernel(u4.reshape(NP, H), src, dst, zeros8)

    return pl.pallas_call(
        _t4_body,
        out_shape=jax.ShapeDtypeStruct((B, H), _f32),
    )(z4.reshape(NC, RN, 128), u4, dinv8, wd4, b128[3],
      indices.reshape(B, 1), e16, g8)
